# baseline (speedup 1.0000x reference)
import functools

import jax
import jax.numpy as jnp
from jax import lax
from jax.experimental import pallas as pl
from jax.experimental.pallas import tpu as pltpu
from jax.experimental.pallas import tpu_sc as plsc

DIM = 512
N_TOK = 8192
GRID = 128
PAD = 3
WPAD = GRID + 2 * PAD
HTOT = 140
NROWS = HTOT * WPAD
HB = 16


def _prep_body(loc_ref, idx_ref):
    r = loc_ref[0:1, :]
    c = loc_ref[1:2, :]
    r0 = r - jnp.min(r)
    c0 = c - jnp.min(c)
    idx_ref[0:1, :] = (c0 + PAD) * WPAD + (r0 + PAD)
    idx_ref[1:2, :] = c0 * GRID + r0


def _prep(loc_t):
    return pl.pallas_call(
        _prep_body,
        out_shape=jax.ShapeDtypeStruct((2, N_TOK), jnp.int32),
    )(loc_t)


_NZCH = (NROWS + 63) // 64
_ZFULL = NROWS // 64
_ZTAIL = NROWS - _ZFULL * 64
_ZPER = (_NZCH + 15) // 16


def _scatter_body(idx_hbm, feat_hbm, grid_hbm, idx_v, zbuf, fbuf, sem):
    t = lax.axis_index("s")

    def zrow(r, _):
        for v in range(DIM // 16):
            zbuf[r, pl.ds(v * 16, 16)] = jnp.zeros((16,), jnp.float32)
        return 0
    lax.fori_loop(0, 64, zrow, 0)

    for j in range(_ZPER):
        k = t * _ZPER + j
        @pl.when(k < _ZFULL)
        def _():
            pltpu.sync_copy(zbuf, grid_hbm.at[pl.ds(k * 64, 64)])
        @pl.when(k == _ZFULL)
        def _():
            pltpu.sync_copy(zbuf.at[pl.ds(0, _ZTAIL)],
                            grid_hbm.at[pl.ds(_ZFULL * 64, _ZTAIL)])

    plsc.subcore_barrier()

    pltpu.sync_copy(idx_hbm.at[t], idx_v)
    for j in range(8):
        base = t * 512 + j * 64
        pltpu.sync_copy(feat_hbm.at[pl.ds(base, 64)], fbuf)
        pltpu.async_copy(fbuf, grid_hbm.at[idx_v.at[j]], sem).wait()


def _scatter(idx_s, feat):
    mesh = plsc.VectorSubcoreMesh(core_axis_name="c", subcore_axis_name="s",
                                  num_cores=1)
    return pl.kernel(
        _scatter_body,
        out_type=jax.ShapeDtypeStruct((NROWS, DIM), jnp.float32),
        mesh=mesh,
        scratch_types=[
            pltpu.VMEM((8, 64), jnp.int32),
            pltpu.VMEM((64, DIM), jnp.float32),
            pltpu.VMEM((64, DIM), jnp.float32),
            pltpu.SemaphoreType.DMA,
        ],
    )(idx_s, feat)


def _conv_body(grid_hbm, w_ref, b_ref, g_ref, be_ref, y_ref, sc_ref, sh_ref,
               xbuf, s_acc, ss_acc, sem):
    i = pl.program_id(0)
    copy = pltpu.make_async_copy(grid_hbm.at[pl.ds(i * HB, HB + 6)], xbuf, sem)
    copy.start()
    copy.wait()

    x = xbuf[...]
    y = jnp.broadcast_to(b_ref[0, :], (HB, GRID, DIM))
    for dy in range(7):
        xs = x[:, dy:dy + GRID, :]
        for dx in range(7):
            y = y + xs[dx:dx + HB, :, :] * w_ref[dy, dx, :]
    y_ref[...] = y

    ps = jnp.sum(y, axis=(0, 1))[None, :]
    pss = jnp.sum(y * y, axis=(0, 1))[None, :]

    @pl.when(i == 0)
    def _():
        s_acc[...] = ps
        ss_acc[...] = pss

    @pl.when(i > 0)
    def _():
        s_acc[...] = s_acc[...] + ps
        ss_acc[...] = ss_acc[...] + pss

    n = float(GRID * GRID)
    mean = s_acc[...] / n
    var = ss_acc[...] / n - mean * mean
    scale = g_ref[...] * lax.rsqrt(var + 1e-5)
    sc_ref[...] = scale
    sh_ref[...] = be_ref[...] - mean * scale


def _conv(grid3, wc, bc, gamma, beta):
    nsteps = GRID // HB
    return pl.pallas_call(
        _conv_body,
        grid=(nsteps,),
        in_specs=[
            pl.BlockSpec(memory_space=pl.ANY),
            pl.BlockSpec((7, 7, DIM), lambda i: (0, 0, 0)),
            pl.BlockSpec((1, DIM), lambda i: (0, 0)),
            pl.BlockSpec((1, DIM), lambda i: (0, 0)),
            pl.BlockSpec((1, DIM), lambda i: (0, 0)),
        ],
        out_specs=[
            pl.BlockSpec((HB, GRID, DIM), lambda i: (i, 0, 0)),
            pl.BlockSpec((1, DIM), lambda i: (0, 0)),
            pl.BlockSpec((1, DIM), lambda i: (0, 0)),
        ],
        out_shape=[
            jax.ShapeDtypeStruct((GRID, GRID, DIM), jnp.float32),
            jax.ShapeDtypeStruct((1, DIM), jnp.float32),
            jax.ShapeDtypeStruct((1, DIM), jnp.float32),
        ],
        scratch_shapes=[
            pltpu.VMEM((HB + 6, WPAD, DIM), jnp.float32),
            pltpu.VMEM((1, DIM), jnp.float32),
            pltpu.VMEM((1, DIM), jnp.float32),
            pltpu.SemaphoreType.DMA,
        ],
    )(grid3, wc, bc, gamma, beta)


def _gather_body(y_hbm, idx_hbm, out_hbm, idx_v, buf, sem):
    nc = 2
    w = lax.axis_index("s") * nc + lax.axis_index("c")
    pltpu.sync_copy(idx_hbm.at[w], idx_v)
    for j in range(4):
        pltpu.async_copy(y_hbm.at[idx_v.at[j]], buf, sem).wait()
        pltpu.sync_copy(buf, out_hbm.at[pl.ds(w * 256 + j * 64, 64)])


def _gather(yflat, idx_g):
    mesh = plsc.VectorSubcoreMesh(core_axis_name="c", subcore_axis_name="s")
    return pl.kernel(
        _gather_body,
        out_type=jax.ShapeDtypeStruct((N_TOK, DIM), jnp.float32),
        mesh=mesh,
        scratch_types=[
            pltpu.VMEM((4, 64), jnp.int32),
            pltpu.VMEM((64, DIM), jnp.float32),
            pltpu.SemaphoreType.DMA,
        ],
    )(yflat, idx_g)


def _combine_body(g_ref, f_ref, sc_ref, sh_ref, o_ref):
    o_ref[...] = g_ref[...] * sc_ref[...] + sh_ref[...] + f_ref[...]


def _combine(g, feat, scale, shift):
    nb = 8
    rb = N_TOK // nb
    return pl.pallas_call(
        _combine_body,
        grid=(nb,),
        in_specs=[
            pl.BlockSpec((rb, DIM), lambda i: (i, 0)),
            pl.BlockSpec((rb, DIM), lambda i: (i, 0)),
            pl.BlockSpec((1, DIM), lambda i: (0, 0)),
            pl.BlockSpec((1, DIM), lambda i: (0, 0)),
        ],
        out_specs=pl.BlockSpec((rb, DIM), lambda i: (i, 0)),
        out_shape=jax.ShapeDtypeStruct((N_TOK, DIM), jnp.float32),
    )(g, feat, scale, shift)


@jax.jit
def kernel(x, location, w7, b7, w5, b5, w3, b3, gamma, beta):
    cls = x[0:1]
    feat = x[1:]

    w5p = jnp.pad(w5[:, 0], ((0, 0), (1, 1), (1, 1)))
    w3p = jnp.pad(w3[:, 0], ((0, 0), (2, 2), (2, 2)))
    wc = jnp.transpose(w7[:, 0] + w5p + w3p, (1, 2, 0))
    bc = (b7 + b5 + b3).reshape(1, DIM)

    idx2 = _prep(location.T)
    idx_s = idx2[0].reshape(16, 8, 64)
    idx_g = idx2[1].reshape(32, 4, 64)

    grid_flat = _scatter(idx_s, feat)
    grid3 = grid_flat.reshape(HTOT, WPAD, DIM)

    y, scale, shift = _conv(grid3, wc, bc,
                            gamma.reshape(1, DIM), beta.reshape(1, DIM))
    yflat = y.reshape(GRID * GRID, DIM)

    g = _gather(yflat, idx_g)
    out_feat = _combine(g, feat, scale, shift)
    return jnp.concatenate([cls, out_feat], axis=0)


# speedup vs baseline: 1.3352x; 1.3352x over previous
import functools

import jax
import jax.numpy as jnp
from jax import lax
from jax.experimental import pallas as pl
from jax.experimental.pallas import tpu as pltpu
from jax.experimental.pallas import tpu_sc as plsc

DIM = 512
N_TOK = 8192
GRID = 128
PAD = 3
WPAD = GRID + 2 * PAD
HTOT = 140
NROWS = HTOT * WPAD
HB = 16


def _prep_body(loc_ref, idx_ref):
    r = loc_ref[0:1, :]
    c = loc_ref[1:2, :]
    r0 = r - jnp.min(r)
    c0 = c - jnp.min(c)
    idx_ref[0:1, :] = (c0 + PAD) * WPAD + (r0 + PAD)
    idx_ref[1:2, :] = c0 * GRID + r0


def _prep(loc_t):
    return pl.pallas_call(
        _prep_body,
        out_shape=jax.ShapeDtypeStruct((2, N_TOK), jnp.int32),
    )(loc_t)


_NZCH = (NROWS + 63) // 64
_ZFULL = NROWS // 64
_ZTAIL = NROWS - _ZFULL * 64
_ZPER = (_NZCH + 15) // 16


def _scatter_body(idx_hbm, feat_hbm, grid_hbm, idx_v, zbuf, fbuf, sem):
    t = lax.axis_index("s")

    def zrow(r, _):
        for v in range(DIM // 16):
            zbuf[r, pl.ds(v * 16, 16)] = jnp.zeros((16,), jnp.float32)
        return 0
    lax.fori_loop(0, 64, zrow, 0)

    for j in range(_ZPER):
        k = t * _ZPER + j
        @pl.when(k < _ZFULL)
        def _():
            pltpu.sync_copy(zbuf, grid_hbm.at[pl.ds(k * 64, 64)])
        @pl.when(k == _ZFULL)
        def _():
            pltpu.sync_copy(zbuf.at[pl.ds(0, _ZTAIL)],
                            grid_hbm.at[pl.ds(_ZFULL * 64, _ZTAIL)])

    plsc.subcore_barrier()

    pltpu.sync_copy(idx_hbm.at[t], idx_v)
    for j in range(8):
        base = t * 512 + j * 64
        pltpu.sync_copy(feat_hbm.at[pl.ds(base, 64)], fbuf)
        pltpu.async_copy(fbuf, grid_hbm.at[idx_v.at[j]], sem).wait()


def _scatter(idx_s, feat):
    mesh = plsc.VectorSubcoreMesh(core_axis_name="c", subcore_axis_name="s",
                                  num_cores=1)
    return pl.kernel(
        _scatter_body,
        out_type=jax.ShapeDtypeStruct((NROWS, DIM), jnp.float32),
        mesh=mesh,
        scratch_types=[
            pltpu.VMEM((8, 64), jnp.int32),
            pltpu.VMEM((64, DIM), jnp.float32),
            pltpu.VMEM((64, DIM), jnp.float32),
            pltpu.SemaphoreType.DMA,
        ],
    )(idx_s, feat)


def _conv_body(grid_hbm, w_ref, b_ref, g_ref, be_ref, y_ref, sc_ref, sh_ref,
               xbuf, xs_buf, s_acc, ss_acc, sem):
    i = pl.program_id(0)
    copy = pltpu.make_async_copy(grid_hbm.at[pl.ds(i * HB, HB + 6)], xbuf, sem)
    copy.start()
    copy.wait()

    for dy in range(7):
        xs_buf[...] = xbuf[:, dy:dy + GRID, :]
        xsv = xs_buf[...]
        p = xsv[0:HB, :, :] * w_ref[dy, 0, :]
        for dx in range(1, 7):
            p = p + xsv[dx:dx + HB, :, :] * w_ref[dy, dx, :]
        if dy == 0:
            y_ref[...] = p + b_ref[0, :]
        else:
            y_ref[...] = y_ref[...] + p
    y = y_ref[...]

    ps = jnp.sum(y, axis=(0, 1))[None, :]
    pss = jnp.sum(y * y, axis=(0, 1))[None, :]

    @pl.when(i == 0)
    def _():
        s_acc[...] = ps
        ss_acc[...] = pss

    @pl.when(i > 0)
    def _():
        s_acc[...] = s_acc[...] + ps
        ss_acc[...] = ss_acc[...] + pss

    n = float(GRID * GRID)
    mean = s_acc[...] / n
    var = ss_acc[...] / n - mean * mean
    scale = g_ref[...] * lax.rsqrt(var + 1e-5)
    sc_ref[...] = scale
    sh_ref[...] = be_ref[...] - mean * scale


def _conv(grid3, wc, bc, gamma, beta):
    nsteps = GRID // HB
    return pl.pallas_call(
        _conv_body,
        grid=(nsteps,),
        in_specs=[
            pl.BlockSpec(memory_space=pl.ANY),
            pl.BlockSpec((7, 7, DIM), lambda i: (0, 0, 0)),
            pl.BlockSpec((1, DIM), lambda i: (0, 0)),
            pl.BlockSpec((1, DIM), lambda i: (0, 0)),
            pl.BlockSpec((1, DIM), lambda i: (0, 0)),
        ],
        out_specs=[
            pl.BlockSpec((HB, GRID, DIM), lambda i: (i, 0, 0)),
            pl.BlockSpec((1, DIM), lambda i: (0, 0)),
            pl.BlockSpec((1, DIM), lambda i: (0, 0)),
        ],
        out_shape=[
            jax.ShapeDtypeStruct((GRID, GRID, DIM), jnp.float32),
            jax.ShapeDtypeStruct((1, DIM), jnp.float32),
            jax.ShapeDtypeStruct((1, DIM), jnp.float32),
        ],
        scratch_shapes=[
            pltpu.VMEM((HB + 6, WPAD, DIM), jnp.float32),
            pltpu.VMEM((HB + 6, GRID, DIM), jnp.float32),
            pltpu.VMEM((1, DIM), jnp.float32),
            pltpu.VMEM((1, DIM), jnp.float32),
            pltpu.SemaphoreType.DMA,
        ],
    )(grid3, wc, bc, gamma, beta)


def _gather_body(y_hbm, idx_hbm, out_hbm, idx_v, buf, sem):
    nc = 2
    w = lax.axis_index("s") * nc + lax.axis_index("c")
    pltpu.sync_copy(idx_hbm.at[w], idx_v)
    for j in range(4):
        pltpu.async_copy(y_hbm.at[idx_v.at[j]], buf, sem).wait()
        pltpu.sync_copy(buf, out_hbm.at[pl.ds(w * 256 + j * 64, 64)])


def _gather(yflat, idx_g):
    mesh = plsc.VectorSubcoreMesh(core_axis_name="c", subcore_axis_name="s")
    return pl.kernel(
        _gather_body,
        out_type=jax.ShapeDtypeStruct((N_TOK, DIM), jnp.float32),
        mesh=mesh,
        scratch_types=[
            pltpu.VMEM((4, 64), jnp.int32),
            pltpu.VMEM((64, DIM), jnp.float32),
            pltpu.SemaphoreType.DMA,
        ],
    )(yflat, idx_g)


def _combine_body(g_ref, f_ref, sc_ref, sh_ref, o_ref):
    o_ref[...] = g_ref[...] * sc_ref[...] + sh_ref[...] + f_ref[...]


def _combine(g, feat, scale, shift):
    nb = 8
    rb = N_TOK // nb
    return pl.pallas_call(
        _combine_body,
        grid=(nb,),
        in_specs=[
            pl.BlockSpec((rb, DIM), lambda i: (i, 0)),
            pl.BlockSpec((rb, DIM), lambda i: (i, 0)),
            pl.BlockSpec((1, DIM), lambda i: (0, 0)),
            pl.BlockSpec((1, DIM), lambda i: (0, 0)),
        ],
        out_specs=pl.BlockSpec((rb, DIM), lambda i: (i, 0)),
        out_shape=jax.ShapeDtypeStruct((N_TOK, DIM), jnp.float32),
    )(g, feat, scale, shift)


@jax.jit
def kernel(x, location, w7, b7, w5, b5, w3, b3, gamma, beta):
    cls = x[0:1]
    feat = x[1:]

    w5p = jnp.pad(w5[:, 0], ((0, 0), (1, 1), (1, 1)))
    w3p = jnp.pad(w3[:, 0], ((0, 0), (2, 2), (2, 2)))
    wc = jnp.transpose(w7[:, 0] + w5p + w3p, (1, 2, 0))
    bc = (b7 + b5 + b3).reshape(1, DIM)

    idx2 = _prep(location.T)
    idx_s = idx2[0].reshape(16, 8, 64)
    idx_g = idx2[1].reshape(32, 4, 64)

    grid_flat = _scatter(idx_s, feat)
    grid3 = grid_flat.reshape(HTOT, WPAD, DIM)

    y, scale, shift = _conv(grid3, wc, bc,
                            gamma.reshape(1, DIM), beta.reshape(1, DIM))
    yflat = y.reshape(GRID * GRID, DIM)

    g = _gather(yflat, idx_g)
    out_feat = _combine(g, feat, scale, shift)
    return jnp.concatenate([cls, out_feat], axis=0)


# speedup vs baseline: 1.4572x; 1.0914x over previous
import functools

import jax
import jax.numpy as jnp
from jax import lax
from jax.experimental import pallas as pl
from jax.experimental.pallas import tpu as pltpu
from jax.experimental.pallas import tpu_sc as plsc

DIM = 512
N_TOK = 8192
GRID = 128
PAD = 3
WPAD = GRID + 2 * PAD
HTOT = 140
NROWS = HTOT * WPAD
HB = 16


def _prep_body(loc_ref, idx_ref):
    r = loc_ref[0:1, :]
    c = loc_ref[1:2, :]
    r0 = r - jnp.min(r)
    c0 = c - jnp.min(c)
    idx_ref[0:1, :] = (c0 + PAD) * WPAD + (r0 + PAD)
    idx_ref[1:2, :] = c0 * GRID + r0


def _prep(loc_t):
    return pl.pallas_call(
        _prep_body,
        out_shape=jax.ShapeDtypeStruct((2, N_TOK), jnp.int32),
    )(loc_t)


_NZCH = (NROWS + 63) // 64
_ZFULL = NROWS // 64
_ZTAIL = NROWS - _ZFULL * 64
_ZPER = (_NZCH + 15) // 16


def _scatter_body(idx_hbm, feat_hbm, grid_hbm, idx_v, zbuf, fbuf, sem):
    t = lax.axis_index("s")

    def zrow(r, _):
        for v in range(DIM // 16):
            zbuf[r, pl.ds(v * 16, 16)] = jnp.zeros((16,), jnp.float32)
        return 0
    lax.fori_loop(0, 64, zrow, 0)

    for j in range(_ZPER):
        k = t * _ZPER + j
        @pl.when(k < _ZFULL)
        def _():
            pltpu.sync_copy(zbuf, grid_hbm.at[pl.ds(k * 64, 64)])
        @pl.when(k == _ZFULL)
        def _():
            pltpu.sync_copy(zbuf.at[pl.ds(0, _ZTAIL)],
                            grid_hbm.at[pl.ds(_ZFULL * 64, _ZTAIL)])

    plsc.subcore_barrier()

    pltpu.sync_copy(idx_hbm.at[t], idx_v)
    for j in range(8):
        base = t * 512 + j * 64
        pltpu.sync_copy(feat_hbm.at[pl.ds(base, 64)], fbuf)
        pltpu.async_copy(fbuf, grid_hbm.at[idx_v.at[j]], sem).wait()


def _scatter(idx_s, feat):
    mesh = plsc.VectorSubcoreMesh(core_axis_name="c", subcore_axis_name="s",
                                  num_cores=1)
    return pl.kernel(
        _scatter_body,
        out_type=jax.ShapeDtypeStruct((NROWS, DIM), jnp.float32),
        mesh=mesh,
        scratch_types=[
            pltpu.VMEM((8, 64), jnp.int32),
            pltpu.VMEM((64, DIM), jnp.float32),
            pltpu.VMEM((64, DIM), jnp.float32),
            pltpu.SemaphoreType.DMA,
        ],
    )(idx_s, feat)


def _conv_body(grid_hbm, w_ref, b_ref, g_ref, be_ref, y_ref, sc_ref, sh_ref,
               xbuf, xs_buf, s_acc, ss_acc, sem):
    i = pl.program_id(0)
    nsteps = pl.num_programs(0)
    s = lax.rem(i, 2)
    sn = lax.rem(i + 1, 2)

    @pl.when(i == 0)
    def _():
        pltpu.make_async_copy(grid_hbm.at[pl.ds(0, HB + 6)],
                              xbuf.at[0], sem.at[0]).start()

    @pl.when(i + 1 < nsteps)
    def _():
        pltpu.make_async_copy(grid_hbm.at[pl.ds((i + 1) * HB, HB + 6)],
                              xbuf.at[sn], sem.at[sn]).start()

    pltpu.make_async_copy(grid_hbm.at[pl.ds(i * HB, HB + 6)],
                          xbuf.at[s], sem.at[s]).wait()

    for dy in range(7):
        xs_buf[...] = xbuf[s, :, dy:dy + GRID, :]
        xsv = xs_buf[...]
        p = xsv[0:HB, :, :] * w_ref[dy, 0, :]
        for dx in range(1, 7):
            p = p + xsv[dx:dx + HB, :, :] * w_ref[dy, dx, :]
        if dy == 0:
            y_ref[...] = p + b_ref[0, :]
        else:
            y_ref[...] = y_ref[...] + p
    y = y_ref[...]

    ps = jnp.sum(y, axis=(0, 1))[None, :]
    pss = jnp.sum(y * y, axis=(0, 1))[None, :]

    @pl.when(i == 0)
    def _():
        s_acc[...] = ps
        ss_acc[...] = pss

    @pl.when(i > 0)
    def _():
        s_acc[...] = s_acc[...] + ps
        ss_acc[...] = ss_acc[...] + pss

    n = float(GRID * GRID)
    mean = s_acc[...] / n
    var = ss_acc[...] / n - mean * mean
    scale = g_ref[...] * lax.rsqrt(var + 1e-5)
    sc_ref[...] = scale
    sh_ref[...] = be_ref[...] - mean * scale


def _conv(grid3, wc, bc, gamma, beta):
    nsteps = GRID // HB
    return pl.pallas_call(
        _conv_body,
        grid=(nsteps,),
        in_specs=[
            pl.BlockSpec(memory_space=pl.ANY),
            pl.BlockSpec((7, 7, DIM), lambda i: (0, 0, 0)),
            pl.BlockSpec((1, DIM), lambda i: (0, 0)),
            pl.BlockSpec((1, DIM), lambda i: (0, 0)),
            pl.BlockSpec((1, DIM), lambda i: (0, 0)),
        ],
        out_specs=[
            pl.BlockSpec((HB, GRID, DIM), lambda i: (i, 0, 0)),
            pl.BlockSpec((1, DIM), lambda i: (0, 0)),
            pl.BlockSpec((1, DIM), lambda i: (0, 0)),
        ],
        out_shape=[
            jax.ShapeDtypeStruct((GRID, GRID, DIM), jnp.float32),
            jax.ShapeDtypeStruct((1, DIM), jnp.float32),
            jax.ShapeDtypeStruct((1, DIM), jnp.float32),
        ],
        scratch_shapes=[
            pltpu.VMEM((2, HB + 6, WPAD, DIM), jnp.float32),
            pltpu.VMEM((HB + 6, GRID, DIM), jnp.float32),
            pltpu.VMEM((1, DIM), jnp.float32),
            pltpu.VMEM((1, DIM), jnp.float32),
            pltpu.SemaphoreType.DMA((2,)),
        ],
    )(grid3, wc, bc, gamma, beta)


def _gather_body(y_hbm, idx_hbm, out_hbm, idx_v, buf, sem0, sem1):
    nc = 2
    w = lax.axis_index("s") * nc + lax.axis_index("c")
    sems = [sem0, sem1]
    pltpu.sync_copy(idx_hbm.at[w], idx_v)
    cps = [None] * 4
    cps[0] = pltpu.async_copy(y_hbm.at[idx_v.at[0]], buf.at[0], sems[0])
    for j in range(4):
        if j + 1 < 4:
            cps[j + 1] = pltpu.async_copy(y_hbm.at[idx_v.at[j + 1]],
                                          buf.at[(j + 1) % 2], sems[(j + 1) % 2])
        cps[j].wait()
        pltpu.sync_copy(buf.at[j % 2], out_hbm.at[pl.ds(w * 256 + j * 64, 64)])


def _gather(yflat, idx_g):
    mesh = plsc.VectorSubcoreMesh(core_axis_name="c", subcore_axis_name="s")
    return pl.kernel(
        _gather_body,
        out_type=jax.ShapeDtypeStruct((N_TOK, DIM), jnp.float32),
        mesh=mesh,
        scratch_types=[
            pltpu.VMEM((4, 64), jnp.int32),
            pltpu.VMEM((2, 64, DIM), jnp.float32),
            pltpu.SemaphoreType.DMA,
            pltpu.SemaphoreType.DMA,
        ],
    )(yflat, idx_g)


def _combine_body(g_ref, f_ref, sc_ref, sh_ref, o_ref):
    o_ref[...] = g_ref[...] * sc_ref[...] + sh_ref[...] + f_ref[...]


def _combine(g, feat, scale, shift):
    nb = 8
    rb = N_TOK // nb
    return pl.pallas_call(
        _combine_body,
        grid=(nb,),
        in_specs=[
            pl.BlockSpec((rb, DIM), lambda i: (i, 0)),
            pl.BlockSpec((rb, DIM), lambda i: (i, 0)),
            pl.BlockSpec((1, DIM), lambda i: (0, 0)),
            pl.BlockSpec((1, DIM), lambda i: (0, 0)),
        ],
        out_specs=pl.BlockSpec((rb, DIM), lambda i: (i, 0)),
        out_shape=jax.ShapeDtypeStruct((N_TOK, DIM), jnp.float32),
    )(g, feat, scale, shift)


@jax.jit
def kernel(x, location, w7, b7, w5, b5, w3, b3, gamma, beta):
    cls = x[0:1]
    feat = x[1:]

    w5p = jnp.pad(w5[:, 0], ((0, 0), (1, 1), (1, 1)))
    w3p = jnp.pad(w3[:, 0], ((0, 0), (2, 2), (2, 2)))
    wc = jnp.transpose(w7[:, 0] + w5p + w3p, (1, 2, 0))
    bc = (b7 + b5 + b3).reshape(1, DIM)

    idx2 = _prep(location.T)
    idx_s = idx2[0].reshape(16, 8, 64)
    idx_g = idx2[1].reshape(32, 4, 64)

    grid_flat = _scatter(idx_s, feat)
    grid3 = grid_flat.reshape(HTOT, WPAD, DIM)

    y, scale, shift = _conv(grid3, wc, bc,
                            gamma.reshape(1, DIM), beta.reshape(1, DIM))
    yflat = y.reshape(GRID * GRID, DIM)

    g = _gather(yflat, idx_g)
    out_feat = _combine(g, feat, scale, shift)
    return jnp.concatenate([cls, out_feat], axis=0)
